# baseline (device time: 138366 ns/iter reference)
import jax
import jax.numpy as jnp
from jax import lax
from jax.experimental import pallas as pl
from jax.experimental.pallas import tpu as pltpu

N_DEV = 4
SQ = 1024
KV = 1152
HL = 8
DH = 128
HD = HL * DH
BLK = SQ // N_DEV
CH = 256
NCH = 4
WIN = 512
WSTART = (0, 128, 384, 640)
WOFF = (0, 128, 128, 128)
SCALE = 0.08838834764831843
QS = 32.0
NEG = -1e9

_ANY = pltpu.MemorySpace.HBM
_CP = getattr(pltpu, "CompilerParams", None) or getattr(pltpu, "TPUCompilerParams")

_STEPS = [(t, c) for c in range(NCH) for t in range(2)]


def _body(x_ref, wq_ref, kb_ref, vb_ref, wo_ref, out_ref,
          kbuf, vbuf, stg, kq, vq, ctx_ref, pbuf, rs_buf, ag_out, ag_buf,
          stg_sem, a_send0, a_recv0, a_send1, a_recv1,
          rs_ssem, rs_rsem, ag_ssem, ag_rsem):
    me = lax.axis_index("i")

    bsem = pltpu.get_barrier_semaphore()
    for d in range(1, N_DEV):
        pl.semaphore_signal(bsem, inc=1, device_id=((me + d) % N_DEV,),
                            device_id_type=pl.DeviceIdType.MESH)
    pl.semaphore_wait(bsem, N_DEV - 1)

    hbm = (kb_ref, vb_ref)
    qdst = (kq, vq)
    dbuf = (kbuf, vbuf)

    def stg_dma(si):
        t, c = _STEPS[si]
        return pltpu.make_async_copy(
            hbm[t].at[pl.ds(CH * c, CH), :], stg.at[si % 2], stg_sem.at[si % 2])

    def quantize(si):
        return jnp.clip(jnp.round(stg[si % 2] * QS), -127.0, 127.0) \
            .astype(jnp.int8)

    def dev0_rdma(c, t, n, dest):
        return pltpu.make_async_remote_copy(
            src_ref=qdst[t].at[pl.ds(CH * c, CH), pl.ds(HD * dest, HD)],
            dst_ref=dbuf[t].at[pl.ds(CH * c, CH)],
            send_sem=a_send0.at[c, t, n],
            recv_sem=a_recv0.at[c, t],
            device_id=(dest,),
            device_id_type=pl.DeviceIdType.MESH,
        )

    def dev1_rdma(t, n, dest):
        return pltpu.make_async_remote_copy(
            src_ref=qdst[t].at[pl.ds(0, 128), pl.ds(HD * dest, HD)],
            dst_ref=dbuf[t].at[pl.ds(1024, 128)],
            send_sem=a_send1.at[n, t],
            recv_sem=a_recv1.at[t],
            device_id=(dest,),
            device_id_type=pl.DeviceIdType.MESH,
        )

    @pl.when(me == 0)
    def _():
        stg_dma(0).start()
        stg_dma(1).start()
        for si in range(2 * NCH):
            t, c = _STEPS[si]
            stg_dma(si).wait()
            qv = quantize(si)
            qdst[t][CH * c:CH * (c + 1), :] = qv
            dbuf[t][CH * c:CH * (c + 1), :] = qv[:, 0:HD]
            if si < 2 * NCH - 2:
                stg_dma(si + 2).start()
            for n, dest in enumerate((1, 2, 3)):
                dev0_rdma(c, t, n, dest).start()

    @pl.when(me == 1)
    def _():
        stg_dma(0).start()
        stg_dma(1).start()
        for si in range(2):
            t, c = _STEPS[si]
            stg_dma(si).wait()
            qv = quantize(si)
            qdst[t][0:CH, :] = qv
            dbuf[t][1024:1152, :] = qv[0:128, HD:2 * HD]
            for n, dest in enumerate((0, 2, 3)):
                dev1_rdma(t, n, dest).start()

    q = lax.dot_general(x_ref[...], wq_ref[...], (((1,), (0,)), ((), ())),
                        preferred_element_type=jnp.float32).astype(jnp.bfloat16)

    qi = lax.broadcasted_iota(jnp.int32, (BLK, WIN), 0)
    kj = lax.broadcasted_iota(jnp.int32, (BLK, WIN), 1)
    masks = [jnp.where(jnp.abs(qi + off - kj) <= 128, 0.0, NEG).astype(jnp.float32)
             for off in (0, 128)]

    for qb in range(N_DEV):
        if qb == 0:
            @pl.when(me != 0)
            def _():
                for c in range(2):
                    for t in range(2):
                        dev0_rdma(c, t, 0, 1).wait_recv()
        elif qb in (1, 2):
            c = qb + 1
            @pl.when(me != 0)
            def _():
                for t in range(2):
                    dev0_rdma(c, t, 0, 1).wait_recv()
        else:
            @pl.when(me != 1)
            def _():
                for t in range(2):
                    dev1_rdma(t, 0, 0).wait_recv()

        ws = WSTART[qb]
        off = masks[0] if WOFF[qb] == 0 else masks[1]
        for h in range(HL):
            qh = q[qb * BLK:(qb + 1) * BLK, h * DH:(h + 1) * DH]
            kh = kbuf[ws:ws + WIN, h * DH:(h + 1) * DH].astype(jnp.bfloat16)
            s = lax.dot_general(qh, kh, (((1,), (1,)), ((), ())),
                                preferred_element_type=jnp.float32)
            s = s * (SCALE / QS) + off
            m = jnp.max(s, axis=1, keepdims=True)
            e = jnp.exp(s - m)
            r = jnp.sum(e, axis=1, keepdims=True)
            p = (e / r).astype(jnp.bfloat16)
            vh = vbuf[ws:ws + WIN, h * DH:(h + 1) * DH].astype(jnp.bfloat16)
            c_ = lax.dot_general(p, vh, (((1,), (0,)), ((), ())),
                                 preferred_element_type=jnp.float32) * (1.0 / QS)
            ctx_ref[qb * BLK:(qb + 1) * BLK, h * DH:(h + 1) * DH] = \
                c_.astype(jnp.bfloat16)

        pblk = lax.dot_general(ctx_ref[qb * BLK:(qb + 1) * BLK, :], wo_ref[...],
                               (((1,), (0,)), ((), ())),
                               preferred_element_type=jnp.float32)
        pbuf[qb * BLK:(qb + 1) * BLK, :] = pblk.astype(jnp.bfloat16)

        @pl.when(me != qb)
        def _():
            pltpu.make_async_remote_copy(
                src_ref=pbuf.at[pl.ds(qb * BLK, BLK)],
                dst_ref=rs_buf.at[me],
                send_sem=rs_ssem.at[qb],
                recv_sem=rs_rsem.at[me],
                device_id=(qb,),
                device_id_type=pl.DeviceIdType.MESH,
            ).start()

        @pl.when(me == qb)
        def _():
            rs_buf[qb, :, :] = pblk.astype(jnp.bfloat16)

    for s_ in range(N_DEV):
        @pl.when(me != s_)
        def _(s_=s_):
            pltpu.make_async_remote_copy(
                src_ref=pbuf.at[pl.ds(0, BLK)],
                dst_ref=rs_buf.at[s_],
                send_sem=rs_ssem.at[0],
                recv_sem=rs_rsem.at[s_],
                device_id=(0,),
                device_id_type=pl.DeviceIdType.MESH,
            ).wait_recv()

    acc = rs_buf[0].astype(jnp.float32)
    for s_ in range(1, N_DEV):
        acc = acc + rs_buf[s_].astype(jnp.float32)

    out_ref[pl.ds(me * BLK, BLK), :] = acc
    ag_out[...] = acc.astype(jnp.bfloat16)

    def ag_rdma(d):
        dest = (me + d) % N_DEV
        return pltpu.make_async_remote_copy(
            src_ref=ag_out,
            dst_ref=ag_buf.at[3 - d],
            send_sem=ag_ssem.at[d - 1],
            recv_sem=ag_rsem.at[3 - d],
            device_id=(dest,),
            device_id_type=pl.DeviceIdType.MESH,
        )

    for d in range(1, N_DEV):
        ag_rdma(d).start()

    for s_ in range(N_DEV - 1):
        pltpu.make_async_remote_copy(
            src_ref=ag_out,
            dst_ref=ag_buf.at[s_],
            send_sem=ag_ssem.at[0],
            recv_sem=ag_rsem.at[s_],
            device_id=(0,),
            device_id_type=pl.DeviceIdType.MESH,
        ).wait_recv()
        src_pos = (me + s_ + 1) % N_DEV
        out_ref[pl.ds(src_pos * BLK, BLK), :] = ag_buf[s_].astype(jnp.float32)

    @pl.when(me == 0)
    def _():
        for c in range(NCH):
            for t in range(2):
                for n, dest in enumerate((1, 2, 3)):
                    dev0_rdma(c, t, n, dest).wait_send()

    @pl.when(me == 1)
    def _():
        for t in range(2):
            for n, dest in enumerate((0, 2, 3)):
                dev1_rdma(t, n, dest).wait_send()

    for qb in range(N_DEV):
        @pl.when(me != qb)
        def _(qb=qb):
            pltpu.make_async_remote_copy(
                src_ref=pbuf.at[pl.ds(qb * BLK, BLK)],
                dst_ref=rs_buf.at[me],
                send_sem=rs_ssem.at[qb],
                recv_sem=rs_rsem.at[me],
                device_id=(qb,),
                device_id_type=pl.DeviceIdType.MESH,
            ).wait_send()

    for d in range(1, N_DEV):
        ag_rdma(d).wait_send()


def kernel(x, Wq, K_ext, V_ext, Wo):
    xb = x[0].astype(jnp.bfloat16)
    wq = Wq.astype(jnp.bfloat16)
    kb = K_ext[0].reshape(1024, 32 * DH)
    vb = V_ext[0].reshape(1024, 32 * DH)
    wo = Wo.astype(jnp.bfloat16)

    out = pl.pallas_call(
        _body,
        out_shape=jax.ShapeDtypeStruct((SQ, 1024), jnp.float32),
        in_specs=[
            pl.BlockSpec(memory_space=pltpu.VMEM),
            pl.BlockSpec(memory_space=pltpu.VMEM),
            pl.BlockSpec(memory_space=_ANY),
            pl.BlockSpec(memory_space=_ANY),
            pl.BlockSpec(memory_space=pltpu.VMEM),
        ],
        out_specs=pl.BlockSpec(memory_space=pltpu.VMEM),
        scratch_shapes=[
            pltpu.VMEM((KV, HD), jnp.int8),
            pltpu.VMEM((KV, HD), jnp.int8),
            pltpu.VMEM((2, CH, 32 * DH), jnp.float32),
            pltpu.VMEM((1024, 32 * DH), jnp.int8),
            pltpu.VMEM((1024, 32 * DH), jnp.int8),
            pltpu.VMEM((SQ, HD), jnp.bfloat16),
            pltpu.VMEM((SQ, 1024), jnp.bfloat16),
            pltpu.VMEM((N_DEV, BLK, 1024), jnp.bfloat16),
            pltpu.VMEM((BLK, 1024), jnp.bfloat16),
            pltpu.VMEM((3, BLK, 1024), jnp.bfloat16),
            pltpu.SemaphoreType.DMA((2,)),
            pltpu.SemaphoreType.DMA((NCH, 2, 3)),
            pltpu.SemaphoreType.DMA((NCH, 2)),
            pltpu.SemaphoreType.DMA((3, 2)),
            pltpu.SemaphoreType.DMA((2,)),
            pltpu.SemaphoreType.DMA((N_DEV,)),
            pltpu.SemaphoreType.DMA((N_DEV,)),
            pltpu.SemaphoreType.DMA((3,)),
            pltpu.SemaphoreType.DMA((3,)),
        ],
        compiler_params=_CP(collective_id=0),
    )(xb, wq, kb, vb, wo)
    return out.reshape(1, SQ, 1024)


# device time: 108649 ns/iter; 1.2735x vs baseline; 1.2735x over previous
import jax
import jax.numpy as jnp
from jax import lax
from jax.experimental import pallas as pl
from jax.experimental.pallas import tpu as pltpu

N_DEV = 4
SQ = 1024
KV = 1152
HL = 8
DH = 128
HD = HL * DH
BLK = SQ // N_DEV
CH = 256
NCH = 4
WIN = 512
WSTART = (0, 128, 384, 640)
WOFF = (0, 128, 128, 128)
SCALE = 0.08838834764831843
QS = 32.0
NEG = -1e9

_ANY = pltpu.MemorySpace.HBM
_CP = getattr(pltpu, "CompilerParams", None) or getattr(pltpu, "TPUCompilerParams")

_STEPS = [(t, c) for c in range(NCH) for t in range(2)]


def _body(x_ref, wq_ref, kb_ref, vb_ref, wo_ref, out_ref,
          kbuf, vbuf, ctx_ref, pbuf, rs_buf, ag_out, ag_buf,
          a_send0, a_recv0, a_send1, a_recv1, loc_sem,
          rs_ssem, rs_rsem, ag_ssem, ag_rsem):
    me = lax.axis_index("i")

    bsem = pltpu.get_barrier_semaphore()
    for d in range(1, N_DEV):
        pl.semaphore_signal(bsem, inc=1, device_id=((me + d) % N_DEV,),
                            device_id_type=pl.DeviceIdType.MESH)
    pl.semaphore_wait(bsem, N_DEV - 1)

    srcs = ((kb_ref, kbuf), (vb_ref, vbuf))

    def dev0_rdma(c, t, n, dest):
        src_ref, dstbuf = srcs[t]
        return pltpu.make_async_remote_copy(
            src_ref=src_ref.at[pl.ds(CH * c, CH), pl.ds(HD * dest, HD)],
            dst_ref=dstbuf.at[pl.ds(CH * c, CH)],
            send_sem=a_send0.at[c, t, n],
            recv_sem=a_recv0.at[c, t],
            device_id=(dest,),
            device_id_type=pl.DeviceIdType.MESH,
        )

    def dev1_rdma(t, n, dest):
        src_ref, dstbuf = srcs[t]
        return pltpu.make_async_remote_copy(
            src_ref=src_ref.at[pl.ds(0, 128), pl.ds(HD * dest, HD)],
            dst_ref=dstbuf.at[pl.ds(1024, 128)],
            send_sem=a_send1.at[n, t],
            recv_sem=a_recv1.at[t],
            device_id=(dest,),
            device_id_type=pl.DeviceIdType.MESH,
        )

    def loc_copy_dev0(t):
        src_ref, dstbuf = srcs[t]
        return pltpu.make_async_copy(
            src_ref.at[:, pl.ds(0, HD)], dstbuf.at[pl.ds(0, 1024)], loc_sem.at[t])

    def loc_copy_dev1(t):
        src_ref, dstbuf = srcs[t]
        return pltpu.make_async_copy(
            src_ref.at[pl.ds(0, 128), pl.ds(HD, HD)],
            dstbuf.at[pl.ds(1024, 128)], loc_sem.at[t])

    @pl.when(me == 0)
    def _():
        for c in range(NCH):
            for t in range(2):
                for n, dest in enumerate((1, 2, 3)):
                    dev0_rdma(c, t, n, dest).start()
        for t in range(2):
            loc_copy_dev0(t).start()

    @pl.when(me == 1)
    def _():
        for t in range(2):
            for n, dest in enumerate((0, 2, 3)):
                dev1_rdma(t, n, dest).start()
        for t in range(2):
            loc_copy_dev1(t).start()

    q = lax.dot_general(x_ref[...], wq_ref[...], (((1,), (0,)), ((), ())),
                        preferred_element_type=jnp.float32).astype(jnp.bfloat16)

    qi = lax.broadcasted_iota(jnp.int32, (BLK, WIN), 0)
    kj = lax.broadcasted_iota(jnp.int32, (BLK, WIN), 1)
    masks = [jnp.where(jnp.abs(qi + off - kj) <= 128, 0.0, NEG).astype(jnp.float32)
             for off in (0, 128)]

    for qb in range(N_DEV):
        if qb == 0:
            @pl.when(me == 0)
            def _():
                for t in range(2):
                    loc_copy_dev0(t).wait()

            @pl.when(me != 0)
            def _():
                for c in range(2):
                    for t in range(2):
                        dev0_rdma(c, t, 0, 1).wait_recv()
        elif qb in (1, 2):
            c = qb + 1
            @pl.when(me != 0)
            def _():
                for t in range(2):
                    dev0_rdma(c, t, 0, 1).wait_recv()
        else:
            @pl.when(me == 1)
            def _():
                for t in range(2):
                    loc_copy_dev1(t).wait()

            @pl.when(me != 1)
            def _():
                for t in range(2):
                    dev1_rdma(t, 0, 0).wait_recv()

        ws = WSTART[qb]
        off = masks[0] if WOFF[qb] == 0 else masks[1]
        for h in range(HL):
            qh = q[qb * BLK:(qb + 1) * BLK, h * DH:(h + 1) * DH]
            kh = kbuf[ws:ws + WIN, h * DH:(h + 1) * DH].astype(jnp.bfloat16)
            s = lax.dot_general(qh, kh, (((1,), (1,)), ((), ())),
                                preferred_element_type=jnp.float32)
            s = s * (SCALE / QS) + off
            m = jnp.max(s, axis=1, keepdims=True)
            e = jnp.exp(s - m)
            r = jnp.sum(e, axis=1, keepdims=True)
            p = (e / r).astype(jnp.bfloat16)
            vh = vbuf[ws:ws + WIN, h * DH:(h + 1) * DH].astype(jnp.bfloat16)
            c_ = lax.dot_general(p, vh, (((1,), (0,)), ((), ())),
                                 preferred_element_type=jnp.float32) * (1.0 / QS)
            ctx_ref[qb * BLK:(qb + 1) * BLK, h * DH:(h + 1) * DH] = \
                c_.astype(jnp.bfloat16)

        pblk = lax.dot_general(ctx_ref[qb * BLK:(qb + 1) * BLK, :], wo_ref[...],
                               (((1,), (0,)), ((), ())),
                               preferred_element_type=jnp.float32)
        pbuf[qb * BLK:(qb + 1) * BLK, :] = pblk.astype(jnp.bfloat16)

        @pl.when(me != qb)
        def _():
            pltpu.make_async_remote_copy(
                src_ref=pbuf.at[pl.ds(qb * BLK, BLK)],
                dst_ref=rs_buf.at[me],
                send_sem=rs_ssem.at[qb],
                recv_sem=rs_rsem.at[me],
                device_id=(qb,),
                device_id_type=pl.DeviceIdType.MESH,
            ).start()

        @pl.when(me == qb)
        def _():
            rs_buf[qb, :, :] = pblk.astype(jnp.bfloat16)

    for s_ in range(N_DEV):
        @pl.when(me != s_)
        def _(s_=s_):
            pltpu.make_async_remote_copy(
                src_ref=pbuf.at[pl.ds(0, BLK)],
                dst_ref=rs_buf.at[s_],
                send_sem=rs_ssem.at[0],
                recv_sem=rs_rsem.at[s_],
                device_id=(0,),
                device_id_type=pl.DeviceIdType.MESH,
            ).wait_recv()

    acc = rs_buf[0].astype(jnp.float32)
    for s_ in range(1, N_DEV):
        acc = acc + rs_buf[s_].astype(jnp.float32)

    out_ref[pl.ds(me * BLK, BLK), :] = acc
    ag_out[...] = acc.astype(jnp.bfloat16)

    def ag_rdma(d):
        dest = (me + d) % N_DEV
        return pltpu.make_async_remote_copy(
            src_ref=ag_out,
            dst_ref=ag_buf.at[3 - d],
            send_sem=ag_ssem.at[d - 1],
            recv_sem=ag_rsem.at[3 - d],
            device_id=(dest,),
            device_id_type=pl.DeviceIdType.MESH,
        )

    for d in range(1, N_DEV):
        ag_rdma(d).start()

    for s_ in range(N_DEV - 1):
        pltpu.make_async_remote_copy(
            src_ref=ag_out,
            dst_ref=ag_buf.at[s_],
            send_sem=ag_ssem.at[0],
            recv_sem=ag_rsem.at[s_],
            device_id=(0,),
            device_id_type=pl.DeviceIdType.MESH,
        ).wait_recv()
        src_pos = (me + s_ + 1) % N_DEV
        out_ref[pl.ds(src_pos * BLK, BLK), :] = ag_buf[s_].astype(jnp.float32)

    @pl.when(me == 0)
    def _():
        for c in range(NCH):
            for t in range(2):
                for n, dest in enumerate((1, 2, 3)):
                    dev0_rdma(c, t, n, dest).wait_send()

    @pl.when(me == 1)
    def _():
        for t in range(2):
            for n, dest in enumerate((0, 2, 3)):
                dev1_rdma(t, n, dest).wait_send()

    for qb in range(N_DEV):
        @pl.when(me != qb)
        def _(qb=qb):
            pltpu.make_async_remote_copy(
                src_ref=pbuf.at[pl.ds(qb * BLK, BLK)],
                dst_ref=rs_buf.at[me],
                send_sem=rs_ssem.at[qb],
                recv_sem=rs_rsem.at[me],
                device_id=(qb,),
                device_id_type=pl.DeviceIdType.MESH,
            ).wait_send()

    for d in range(1, N_DEV):
        ag_rdma(d).wait_send()


def kernel(x, Wq, K_ext, V_ext, Wo):
    xb = x[0].astype(jnp.bfloat16)
    wq = Wq.astype(jnp.bfloat16)
    kb = jnp.clip(jnp.round(K_ext[0] * QS), -127, 127).astype(jnp.int8) \
        .reshape(1024, 32 * DH)
    vb = jnp.clip(jnp.round(V_ext[0] * QS), -127, 127).astype(jnp.int8) \
        .reshape(1024, 32 * DH)
    wo = Wo.astype(jnp.bfloat16)

    out = pl.pallas_call(
        _body,
        out_shape=jax.ShapeDtypeStruct((SQ, 1024), jnp.float32),
        in_specs=[
            pl.BlockSpec(memory_space=pltpu.VMEM),
            pl.BlockSpec(memory_space=pltpu.VMEM),
            pl.BlockSpec(memory_space=_ANY),
            pl.BlockSpec(memory_space=_ANY),
            pl.BlockSpec(memory_space=pltpu.VMEM),
        ],
        out_specs=pl.BlockSpec(memory_space=pltpu.VMEM),
        scratch_shapes=[
            pltpu.VMEM((KV, HD), jnp.int8),
            pltpu.VMEM((KV, HD), jnp.int8),
            pltpu.VMEM((SQ, HD), jnp.bfloat16),
            pltpu.VMEM((SQ, 1024), jnp.bfloat16),
            pltpu.VMEM((N_DEV, BLK, 1024), jnp.bfloat16),
            pltpu.VMEM((BLK, 1024), jnp.bfloat16),
            pltpu.VMEM((3, BLK, 1024), jnp.bfloat16),
            pltpu.SemaphoreType.DMA((NCH, 2, 3)),
            pltpu.SemaphoreType.DMA((NCH, 2)),
            pltpu.SemaphoreType.DMA((3, 2)),
            pltpu.SemaphoreType.DMA((2,)),
            pltpu.SemaphoreType.DMA((2,)),
            pltpu.SemaphoreType.DMA((N_DEV,)),
            pltpu.SemaphoreType.DMA((N_DEV,)),
            pltpu.SemaphoreType.DMA((3,)),
            pltpu.SemaphoreType.DMA((3,)),
        ],
        compiler_params=_CP(collective_id=0),
    )(xb, wq, kb, vb, wo)
    return out.reshape(1, SQ, 1024)


# device time: 106225 ns/iter; 1.3026x vs baseline; 1.0228x over previous
import jax
import jax.numpy as jnp
from jax import lax
from jax.experimental import pallas as pl
from jax.experimental.pallas import tpu as pltpu

N_DEV = 4
SQ = 1024
KV = 1152
HL = 8
DH = 128
HD = HL * DH
BLK = SQ // N_DEV
CH = 256
NCH = 4
WIN = 512
WSTART = (0, 128, 384, 640)
WOFF = (0, 128, 128, 128)
SCALE = 0.08838834764831843
QS = 32.0
NEG = -1e9

_ANY = pltpu.MemorySpace.HBM
_CP = getattr(pltpu, "CompilerParams", None) or getattr(pltpu, "TPUCompilerParams")

_STEPS = [(t, c) for c in range(NCH) for t in range(2)]


def _body(x_ref, wq_ref, kb_ref, vb_ref, wo_ref, out_ref,
          kbuf, vbuf, kvm, vvm, ctx_ref, pbuf, rs_buf, ag_out, ag_buf,
          stg_sem, a_send0, a_recv0, a_send1, a_recv1, loc_sem,
          rs_ssem, rs_rsem, ag_ssem, ag_rsem):
    me = lax.axis_index("i")

    bsem = pltpu.get_barrier_semaphore()
    for d in range(1, N_DEV):
        pl.semaphore_signal(bsem, inc=1, device_id=((me + d) % N_DEV,),
                            device_id_type=pl.DeviceIdType.MESH)
    pl.semaphore_wait(bsem, N_DEV - 1)

    srcs = ((kvm, kbuf), (vvm, vbuf))

    def stage_dma(t, rows):
        hbm = (kb_ref, vb_ref)[t]
        vmm = (kvm, vvm)[t]
        return pltpu.make_async_copy(
            hbm.at[pl.ds(0, rows), :], vmm.at[pl.ds(0, rows)], stg_sem.at[t])

    def dev0_rdma(c, t, n, dest):
        src_ref, dstbuf = srcs[t]
        return pltpu.make_async_remote_copy(
            src_ref=src_ref.at[pl.ds(CH * c, CH), pl.ds(HD * dest, HD)],
            dst_ref=dstbuf.at[pl.ds(CH * c, CH)],
            send_sem=a_send0.at[c, t, n],
            recv_sem=a_recv0.at[c, t],
            device_id=(dest,),
            device_id_type=pl.DeviceIdType.MESH,
        )

    def dev1_rdma(t, n, dest):
        src_ref, dstbuf = srcs[t]
        return pltpu.make_async_remote_copy(
            src_ref=src_ref.at[pl.ds(0, 128), pl.ds(HD * dest, HD)],
            dst_ref=dstbuf.at[pl.ds(1024, 128)],
            send_sem=a_send1.at[n, t],
            recv_sem=a_recv1.at[t],
            device_id=(dest,),
            device_id_type=pl.DeviceIdType.MESH,
        )

    def loc_copy_dev0(t):
        src_ref, dstbuf = srcs[t]
        return pltpu.make_async_copy(
            src_ref.at[:, pl.ds(0, HD)], dstbuf.at[pl.ds(0, 1024)], loc_sem.at[t])

    def loc_copy_dev1(t):
        src_ref, dstbuf = srcs[t]
        return pltpu.make_async_copy(
            src_ref.at[pl.ds(0, 128), pl.ds(HD, HD)],
            dstbuf.at[pl.ds(1024, 128)], loc_sem.at[t])

    @pl.when(me == 0)
    def _():
        stage_dma(0, 1024).start()
        stage_dma(1, 1024).start()
        for t in range(2):
            stage_dma(t, 1024).wait()
            for c in range(NCH):
                for n, dest in enumerate((1, 2, 3)):
                    dev0_rdma(c, t, n, dest).start()
            loc_copy_dev0(t).start()

    @pl.when(me == 1)
    def _():
        stage_dma(0, 128).start()
        stage_dma(1, 128).start()
        for t in range(2):
            stage_dma(t, 128).wait()
            for n, dest in enumerate((0, 2, 3)):
                dev1_rdma(t, n, dest).start()
            loc_copy_dev1(t).start()

    q = lax.dot_general(x_ref[...], wq_ref[...], (((1,), (0,)), ((), ())),
                        preferred_element_type=jnp.float32).astype(jnp.bfloat16)

    qi = lax.broadcasted_iota(jnp.int32, (BLK, WIN), 0)
    kj = lax.broadcasted_iota(jnp.int32, (BLK, WIN), 1)
    masks = [jnp.where(jnp.abs(qi + off - kj) <= 128, 0.0, NEG).astype(jnp.float32)
             for off in (0, 128)]

    for qb in range(N_DEV):
        if qb == 0:
            @pl.when(me == 0)
            def _():
                for t in range(2):
                    loc_copy_dev0(t).wait()

            @pl.when(me != 0)
            def _():
                for c in range(2):
                    for t in range(2):
                        dev0_rdma(c, t, 0, 1).wait_recv()
        elif qb in (1, 2):
            c = qb + 1
            @pl.when(me != 0)
            def _():
                for t in range(2):
                    dev0_rdma(c, t, 0, 1).wait_recv()
        else:
            @pl.when(me == 1)
            def _():
                for t in range(2):
                    loc_copy_dev1(t).wait()

            @pl.when(me != 1)
            def _():
                for t in range(2):
                    dev1_rdma(t, 0, 0).wait_recv()

        ws = WSTART[qb]
        off = masks[0] if WOFF[qb] == 0 else masks[1]
        for h in range(HL):
            qh = q[qb * BLK:(qb + 1) * BLK, h * DH:(h + 1) * DH]
            kh = kbuf[ws:ws + WIN, h * DH:(h + 1) * DH].astype(jnp.bfloat16)
            s = lax.dot_general(qh, kh, (((1,), (1,)), ((), ())),
                                preferred_element_type=jnp.float32)
            s = s * (SCALE / QS) + off
            m = jnp.max(s, axis=1, keepdims=True)
            e = jnp.exp(s - m)
            r = jnp.sum(e, axis=1, keepdims=True)
            p = (e / r).astype(jnp.bfloat16)
            vh = vbuf[ws:ws + WIN, h * DH:(h + 1) * DH].astype(jnp.bfloat16)
            c_ = lax.dot_general(p, vh, (((1,), (0,)), ((), ())),
                                 preferred_element_type=jnp.float32) * (1.0 / QS)
            ctx_ref[qb * BLK:(qb + 1) * BLK, h * DH:(h + 1) * DH] = \
                c_.astype(jnp.bfloat16)

        pblk = lax.dot_general(ctx_ref[qb * BLK:(qb + 1) * BLK, :], wo_ref[...],
                               (((1,), (0,)), ((), ())),
                               preferred_element_type=jnp.float32)
        pbuf[qb * BLK:(qb + 1) * BLK, :] = pblk.astype(jnp.bfloat16)

        @pl.when(me != qb)
        def _():
            pltpu.make_async_remote_copy(
                src_ref=pbuf.at[pl.ds(qb * BLK, BLK)],
                dst_ref=rs_buf.at[me],
                send_sem=rs_ssem.at[qb],
                recv_sem=rs_rsem.at[me],
                device_id=(qb,),
                device_id_type=pl.DeviceIdType.MESH,
            ).start()

        @pl.when(me == qb)
        def _():
            rs_buf[qb, :, :] = pblk.astype(jnp.bfloat16)

    for s_ in range(N_DEV):
        @pl.when(me != s_)
        def _(s_=s_):
            pltpu.make_async_remote_copy(
                src_ref=pbuf.at[pl.ds(0, BLK)],
                dst_ref=rs_buf.at[s_],
                send_sem=rs_ssem.at[0],
                recv_sem=rs_rsem.at[s_],
                device_id=(0,),
                device_id_type=pl.DeviceIdType.MESH,
            ).wait_recv()

    acc = rs_buf[0].astype(jnp.float32)
    for s_ in range(1, N_DEV):
        acc = acc + rs_buf[s_].astype(jnp.float32)

    out_ref[pl.ds(me * BLK, BLK), :] = acc
    ag_out[...] = acc.astype(jnp.bfloat16)

    def ag_rdma(d):
        dest = (me + d) % N_DEV
        return pltpu.make_async_remote_copy(
            src_ref=ag_out,
            dst_ref=ag_buf.at[3 - d],
            send_sem=ag_ssem.at[d - 1],
            recv_sem=ag_rsem.at[3 - d],
            device_id=(dest,),
            device_id_type=pl.DeviceIdType.MESH,
        )

    for d in range(1, N_DEV):
        ag_rdma(d).start()

    for s_ in range(N_DEV - 1):
        pltpu.make_async_remote_copy(
            src_ref=ag_out,
            dst_ref=ag_buf.at[s_],
            send_sem=ag_ssem.at[0],
            recv_sem=ag_rsem.at[s_],
            device_id=(0,),
            device_id_type=pl.DeviceIdType.MESH,
        ).wait_recv()
        src_pos = (me + s_ + 1) % N_DEV
        out_ref[pl.ds(src_pos * BLK, BLK), :] = ag_buf[s_].astype(jnp.float32)

    @pl.when(me == 0)
    def _():
        for c in range(NCH):
            for t in range(2):
                for n, dest in enumerate((1, 2, 3)):
                    dev0_rdma(c, t, n, dest).wait_send()

    @pl.when(me == 1)
    def _():
        for t in range(2):
            for n, dest in enumerate((0, 2, 3)):
                dev1_rdma(t, n, dest).wait_send()

    for qb in range(N_DEV):
        @pl.when(me != qb)
        def _(qb=qb):
            pltpu.make_async_remote_copy(
                src_ref=pbuf.at[pl.ds(qb * BLK, BLK)],
                dst_ref=rs_buf.at[me],
                send_sem=rs_ssem.at[qb],
                recv_sem=rs_rsem.at[me],
                device_id=(qb,),
                device_id_type=pl.DeviceIdType.MESH,
            ).wait_send()

    for d in range(1, N_DEV):
        ag_rdma(d).wait_send()


def kernel(x, Wq, K_ext, V_ext, Wo):
    xb = x[0].astype(jnp.bfloat16)
    wq = Wq.astype(jnp.bfloat16)
    kb = jnp.clip(jnp.round(K_ext[0] * QS), -127, 127).astype(jnp.int8) \
        .reshape(1024, 32 * DH)
    vb = jnp.clip(jnp.round(V_ext[0] * QS), -127, 127).astype(jnp.int8) \
        .reshape(1024, 32 * DH)
    wo = Wo.astype(jnp.bfloat16)

    out = pl.pallas_call(
        _body,
        out_shape=jax.ShapeDtypeStruct((SQ, 1024), jnp.float32),
        in_specs=[
            pl.BlockSpec(memory_space=pltpu.VMEM),
            pl.BlockSpec(memory_space=pltpu.VMEM),
            pl.BlockSpec(memory_space=_ANY),
            pl.BlockSpec(memory_space=_ANY),
            pl.BlockSpec(memory_space=pltpu.VMEM),
        ],
        out_specs=pl.BlockSpec(memory_space=pltpu.VMEM),
        scratch_shapes=[
            pltpu.VMEM((KV, HD), jnp.int8),
            pltpu.VMEM((KV, HD), jnp.int8),
            pltpu.VMEM((1024, 32 * DH), jnp.int8),
            pltpu.VMEM((1024, 32 * DH), jnp.int8),
            pltpu.VMEM((SQ, HD), jnp.bfloat16),
            pltpu.VMEM((SQ, 1024), jnp.bfloat16),
            pltpu.VMEM((N_DEV, BLK, 1024), jnp.bfloat16),
            pltpu.VMEM((BLK, 1024), jnp.bfloat16),
            pltpu.VMEM((3, BLK, 1024), jnp.bfloat16),
            pltpu.SemaphoreType.DMA((2,)),
            pltpu.SemaphoreType.DMA((NCH, 2, 3)),
            pltpu.SemaphoreType.DMA((NCH, 2)),
            pltpu.SemaphoreType.DMA((3, 2)),
            pltpu.SemaphoreType.DMA((2,)),
            pltpu.SemaphoreType.DMA((2,)),
            pltpu.SemaphoreType.DMA((N_DEV,)),
            pltpu.SemaphoreType.DMA((N_DEV,)),
            pltpu.SemaphoreType.DMA((3,)),
            pltpu.SemaphoreType.DMA((3,)),
        ],
        compiler_params=_CP(collective_id=0),
    )(xb, wq, kb, vb, wo)
    return out.reshape(1, SQ, 1024)


# device time: 104686 ns/iter; 1.3217x vs baseline; 1.0147x over previous
import jax
import jax.numpy as jnp
from jax import lax
from jax.experimental import pallas as pl
from jax.experimental.pallas import tpu as pltpu

N_DEV = 4
SQ = 1024
KV = 1152
HL = 8
DH = 128
HD = HL * DH
BLK = SQ // N_DEV
CH = 256
NCH = 4
WIN = 512
WSTART = (0, 128, 384, 640)
WOFF = (0, 128, 128, 128)
SCALE = 0.08838834764831843
QS = 32.0
NEG = -1e9

_ANY = pltpu.MemorySpace.HBM

PHASE_A = True
PHASE_C = True
PIPELINE_C = False
_CP = getattr(pltpu, "CompilerParams", None) or getattr(pltpu, "TPUCompilerParams")

_STEPS = [(t, c) for c in range(NCH) for t in range(2)]


def _body(x_ref, wq_ref, kb_ref, vb_ref, wo_ref, out_ref,
          kbuf, vbuf, kvm, vvm, ctx_ref, pbuf, rs_buf, ag_out, ag_buf,
          stg_sem, a_send0, a_recv0, a_send1, a_recv1, loc_sem,
          rs_ssem, rs_rsem, ag_ssem, ag_rsem):
    me = lax.axis_index("i")

    bsem = pltpu.get_barrier_semaphore()
    for d in range(1, N_DEV):
        pl.semaphore_signal(bsem, inc=1, device_id=((me + d) % N_DEV,),
                            device_id_type=pl.DeviceIdType.MESH)
    pl.semaphore_wait(bsem, N_DEV - 1)

    srcs = ((kvm, kbuf), (vvm, vbuf))

    def stage_dma(t, rows):
        hbm = (kb_ref, vb_ref)[t]
        vmm = (kvm, vvm)[t]
        return pltpu.make_async_copy(
            hbm.at[pl.ds(0, rows), :], vmm.at[pl.ds(0, rows)], stg_sem.at[t])

    def dev0_rdma(c, t, n, dest):
        src_ref, dstbuf = srcs[t]
        return pltpu.make_async_remote_copy(
            src_ref=src_ref.at[pl.ds(CH * c, CH), pl.ds(HD * dest, HD)],
            dst_ref=dstbuf.at[pl.ds(CH * c, CH)],
            send_sem=a_send0.at[c, t, n],
            recv_sem=a_recv0.at[c, t],
            device_id=(dest,),
            device_id_type=pl.DeviceIdType.MESH,
        )

    def dev1_rdma(t, n, dest):
        src_ref, dstbuf = srcs[t]
        return pltpu.make_async_remote_copy(
            src_ref=src_ref.at[pl.ds(0, 128), pl.ds(HD * dest, HD)],
            dst_ref=dstbuf.at[pl.ds(1024, 128)],
            send_sem=a_send1.at[n, t],
            recv_sem=a_recv1.at[t],
            device_id=(dest,),
            device_id_type=pl.DeviceIdType.MESH,
        )

    def loc_copy_dev0(t):
        src_ref, dstbuf = srcs[t]
        return pltpu.make_async_copy(
            src_ref.at[:, pl.ds(0, HD)], dstbuf.at[pl.ds(0, 1024)], loc_sem.at[t])

    def loc_copy_dev1(t):
        src_ref, dstbuf = srcs[t]
        return pltpu.make_async_copy(
            src_ref.at[pl.ds(0, 128), pl.ds(HD, HD)],
            dstbuf.at[pl.ds(1024, 128)], loc_sem.at[t])

    if PHASE_A:
        @pl.when(me == 0)
        def _():
            stage_dma(0, 1024).start()
            stage_dma(1, 1024).start()
            for t in range(2):
                stage_dma(t, 1024).wait()
                for c in range(NCH):
                    for n, dest in enumerate((1, 2, 3)):
                        dev0_rdma(c, t, n, dest).start()
                loc_copy_dev0(t).start()

        @pl.when(me == 1)
        def _():
            stage_dma(0, 128).start()
            stage_dma(1, 128).start()
            for t in range(2):
                stage_dma(t, 128).wait()
                for n, dest in enumerate((0, 2, 3)):
                    dev1_rdma(t, n, dest).start()
                loc_copy_dev1(t).start()

    q = lax.dot_general(x_ref[...], wq_ref[...], (((1,), (0,)), ((), ())),
                        preferred_element_type=jnp.float32).astype(jnp.bfloat16)

    qi = lax.broadcasted_iota(jnp.int32, (BLK, WIN), 0)
    kj = lax.broadcasted_iota(jnp.int32, (BLK, WIN), 1)
    masks = [jnp.where(jnp.abs(qi + off - kj) <= 128, 0.0, NEG).astype(jnp.float32)
             for off in (0, 128)]

    def ag_rdma(d):
        dest = (me + d) % N_DEV
        return pltpu.make_async_remote_copy(
            src_ref=ag_out,
            dst_ref=ag_buf.at[3 - d],
            send_sem=ag_ssem.at[d - 1],
            recv_sem=ag_rsem.at[3 - d],
            device_id=(dest,),
            device_id_type=pl.DeviceIdType.MESH,
        )

    def owner_acc(b):
        @pl.when(me == b)
        def _():
            for s_ in range(N_DEV):
                if s_ != b:
                    pltpu.make_async_remote_copy(
                        src_ref=pbuf.at[pl.ds(0, BLK)],
                        dst_ref=rs_buf.at[s_],
                        send_sem=rs_ssem.at[0],
                        recv_sem=rs_rsem.at[s_],
                        device_id=(0,),
                        device_id_type=pl.DeviceIdType.MESH,
                    ).wait_recv()
            acc = rs_buf[0].astype(jnp.float32)
            for s_ in range(1, N_DEV):
                acc = acc + rs_buf[s_].astype(jnp.float32)
            out_ref[b * BLK:(b + 1) * BLK, :] = acc
            ag_out[...] = acc.astype(jnp.bfloat16)
            for d in range(1, N_DEV):
                ag_rdma(d).start()

    for qb in range(N_DEV):
        if not PHASE_A:
            pass
        elif qb == 0:
            @pl.when(me == 0)
            def _():
                for t in range(2):
                    loc_copy_dev0(t).wait()

            @pl.when(me != 0)
            def _():
                for c in range(2):
                    for t in range(2):
                        dev0_rdma(c, t, 0, 1).wait_recv()
        elif qb in (1, 2):
            c = qb + 1
            @pl.when(me != 0)
            def _():
                for t in range(2):
                    dev0_rdma(c, t, 0, 1).wait_recv()
        else:
            @pl.when(me == 1)
            def _():
                for t in range(2):
                    loc_copy_dev1(t).wait()

            @pl.when(me != 1)
            def _():
                for t in range(2):
                    dev1_rdma(t, 0, 0).wait_recv()

        ws = WSTART[qb]
        off = masks[0] if WOFF[qb] == 0 else masks[1]
        for h in range(HL):
            qh = q[qb * BLK:(qb + 1) * BLK, h * DH:(h + 1) * DH]
            kh = kbuf[ws:ws + WIN, h * DH:(h + 1) * DH].astype(jnp.bfloat16)
            s = lax.dot_general(qh, kh, (((1,), (1,)), ((), ())),
                                preferred_element_type=jnp.float32)
            s = s * (SCALE / QS) + off
            m = jnp.max(s, axis=1, keepdims=True)
            e = jnp.exp(s - m)
            r = jnp.sum(e, axis=1, keepdims=True)
            p = (e / r).astype(jnp.bfloat16)
            vh = vbuf[ws:ws + WIN, h * DH:(h + 1) * DH].astype(jnp.bfloat16)
            c_ = lax.dot_general(p, vh, (((1,), (0,)), ((), ())),
                                 preferred_element_type=jnp.float32) * (1.0 / QS)
            ctx_ref[qb * BLK:(qb + 1) * BLK, h * DH:(h + 1) * DH] = \
                c_.astype(jnp.bfloat16)

        pblk = lax.dot_general(ctx_ref[qb * BLK:(qb + 1) * BLK, :], wo_ref[...],
                               (((1,), (0,)), ((), ())),
                               preferred_element_type=jnp.float32)
        pbuf[qb * BLK:(qb + 1) * BLK, :] = pblk.astype(jnp.bfloat16)

        if PHASE_C:
            @pl.when(me != qb)
            def _():
                pltpu.make_async_remote_copy(
                    src_ref=pbuf.at[pl.ds(qb * BLK, BLK)],
                    dst_ref=rs_buf.at[me],
                    send_sem=rs_ssem.at[qb],
                    recv_sem=rs_rsem.at[me],
                    device_id=(qb,),
                    device_id_type=pl.DeviceIdType.MESH,
                ).start()

            @pl.when(me == qb)
            def _():
                rs_buf[qb, :, :] = pblk.astype(jnp.bfloat16)

            if PIPELINE_C and qb >= 1:
                owner_acc(qb - 1)
        else:
            out_ref[qb * BLK:(qb + 1) * BLK, :] = pblk

    if PHASE_C:
        for b in range(N_DEV - 1 if PIPELINE_C else 0, N_DEV):
            owner_acc(b)

        for s_ in range(N_DEV - 1):
            pltpu.make_async_remote_copy(
                src_ref=ag_out,
                dst_ref=ag_buf.at[s_],
                send_sem=ag_ssem.at[0],
                recv_sem=ag_rsem.at[s_],
                device_id=(0,),
                device_id_type=pl.DeviceIdType.MESH,
            ).wait_recv()
            src_pos = (me + s_ + 1) % N_DEV
            out_ref[pl.ds(src_pos * BLK, BLK), :] = ag_buf[s_].astype(jnp.float32)

    if PHASE_A:
        @pl.when(me == 0)
        def _():
            for c in range(NCH):
                for t in range(2):
                    for n, dest in enumerate((1, 2, 3)):
                        dev0_rdma(c, t, n, dest).wait_send()

        @pl.when(me == 1)
        def _():
            for t in range(2):
                for n, dest in enumerate((0, 2, 3)):
                    dev1_rdma(t, n, dest).wait_send()

    if PHASE_C:
        for qb in range(N_DEV):
            @pl.when(me != qb)
            def _(qb=qb):
                pltpu.make_async_remote_copy(
                    src_ref=pbuf.at[pl.ds(qb * BLK, BLK)],
                    dst_ref=rs_buf.at[me],
                    send_sem=rs_ssem.at[qb],
                    recv_sem=rs_rsem.at[me],
                    device_id=(qb,),
                    device_id_type=pl.DeviceIdType.MESH,
                ).wait_send()

        for d in range(1, N_DEV):
            ag_rdma(d).wait_send()


def kernel(x, Wq, K_ext, V_ext, Wo):
    xb = x[0].astype(jnp.bfloat16)
    wq = Wq.astype(jnp.bfloat16)
    kb = jnp.clip(jnp.round(K_ext[0] * QS), -127, 127).astype(jnp.int8) \
        .reshape(1024, 32 * DH)
    vb = jnp.clip(jnp.round(V_ext[0] * QS), -127, 127).astype(jnp.int8) \
        .reshape(1024, 32 * DH)
    wo = Wo.astype(jnp.bfloat16)

    out = pl.pallas_call(
        _body,
        out_shape=jax.ShapeDtypeStruct((SQ, 1024), jnp.float32),
        in_specs=[
            pl.BlockSpec(memory_space=pltpu.VMEM),
            pl.BlockSpec(memory_space=pltpu.VMEM),
            pl.BlockSpec(memory_space=_ANY),
            pl.BlockSpec(memory_space=_ANY),
            pl.BlockSpec(memory_space=pltpu.VMEM),
        ],
        out_specs=pl.BlockSpec(memory_space=pltpu.VMEM),
        scratch_shapes=[
            pltpu.VMEM((KV, HD), jnp.int8),
            pltpu.VMEM((KV, HD), jnp.int8),
            pltpu.VMEM((1024, 32 * DH), jnp.int8),
            pltpu.VMEM((1024, 32 * DH), jnp.int8),
            pltpu.VMEM((SQ, HD), jnp.bfloat16),
            pltpu.VMEM((SQ, 1024), jnp.bfloat16),
            pltpu.VMEM((N_DEV, BLK, 1024), jnp.bfloat16),
            pltpu.VMEM((BLK, 1024), jnp.bfloat16),
            pltpu.VMEM((3, BLK, 1024), jnp.bfloat16),
            pltpu.SemaphoreType.DMA((2,)),
            pltpu.SemaphoreType.DMA((NCH, 2, 3)),
            pltpu.SemaphoreType.DMA((NCH, 2)),
            pltpu.SemaphoreType.DMA((3, 2)),
            pltpu.SemaphoreType.DMA((2,)),
            pltpu.SemaphoreType.DMA((2,)),
            pltpu.SemaphoreType.DMA((N_DEV,)),
            pltpu.SemaphoreType.DMA((N_DEV,)),
            pltpu.SemaphoreType.DMA((3,)),
            pltpu.SemaphoreType.DMA((3,)),
        ],
        compiler_params=_CP(collective_id=0),
    )(xb, wq, kb, vb, wo)
    return out.reshape(1, SQ, 1024)
